# Initial kernel scaffold; baseline (speedup 1.0000x reference)
import jax
import jax.numpy as jnp
from jax.experimental import pallas as pl


def kernel(batch, x, pos, c1_W1, c1_b1, c1_g1, c1_be1, c1_W2, c1_b2, c1_g2, c1_be2, c2_W1, c2_b1, c2_g1, c2_be1, c2_W2, c2_b2, c2_g2, c2_be2, c3_W1, c3_b1, c3_g1, c3_be1, c3_W2, c3_b2, c3_g2, c3_be2, l1_W, l1_b, l1_g, l1_be, m1_W, m1_b, m1_g, m1_be, m2_W, m2_b, m2_g, m2_be, m3_W, m3_b):
    raise NotImplementedError("write your pallas kernel here")


# speedup vs baseline: 4.5836x; 4.5836x over previous
import functools

import jax
import jax.numpy as jnp
from jax import lax
from jax.experimental import pallas as pl
from jax.experimental.pallas import tpu as pltpu
from jax.experimental.pallas import tpu_sc as plsc

N = 8192
KNN = 20
NSEG = 8
F = 64
VF = 128
EPS = 1e-5
NB = 256
NBLK = N // NB
E_TOTAL = N * KNN
NEG_INF = float("-inf")
POS_INF = float("inf")


_SC_INFO = plsc.get_sparse_core_info()
_NW = _SC_INFO.num_cores * _SC_INFO.num_subcores
_B_PER_W = E_TOTAL // _NW
_CHUNK = 512
_NCHUNK = _B_PER_W // _CHUNK

_gather_mesh = plsc.VectorSubcoreMesh(core_axis_name="c", subcore_axis_name="s")


@functools.partial(
    pl.kernel,
    mesh=_gather_mesh,
    out_type=jax.ShapeDtypeStruct((E_TOTAL, VF), jnp.float32),
    scratch_types=[
        pltpu.VMEM((_CHUNK,), jnp.int32),
        pltpu.VMEM((_CHUNK, VF), jnp.float32),
        pltpu.SemaphoreType.DMA,
    ],
)
def _sc_gather(table_hbm, idx_hbm, out_hbm, idx_v, rows_v, sem):
    wid = lax.axis_index("s") * _SC_INFO.num_cores + lax.axis_index("c")
    base = wid * _B_PER_W
    for j in range(_NCHUNK):
        off = base + j * _CHUNK
        pltpu.sync_copy(idx_hbm.at[pl.ds(off, _CHUNK)], idx_v)
        pltpu.async_copy(table_hbm.at[idx_v], rows_v, sem).wait()
        pltpu.sync_copy(rows_v, out_hbm.at[pl.ds(off, _CHUNK)])


def _dotT(a, b):
    return lax.dot_general(a, b, (((1,), (1,)), ((), ())),
                           preferred_element_type=jnp.float32)


def _bn_apply(sel, st_ref, g, be, count):
    mu = st_ref[0:1, :] / count
    var = st_ref[1:2, :] / count - mu * mu
    return g * (sel - mu) / jnp.sqrt(var + EPS) + be


def _bn_apply2(sel, ms_ref, vs_ref, g, be, count):
    mu = ms_ref[...] / count
    var = vs_ref[...] / count
    return g * (sel - mu) / jnp.sqrt(var + EPS) + be


def _prep0_kern(x_ref, xpad_ref, n2_ref):
    xb = x_ref[...]
    d = xb.shape[1]
    xpad_ref[...] = jnp.concatenate(
        [xb, jnp.zeros((NB, VF - d), jnp.float32)], axis=1)
    n2_ref[...] = jnp.sum(xb * xb, axis=1, keepdims=True)


def _next_prep_kern(mx_ref, mn_ref, ms_ref, vs_ref, g_ref, be_ref,
                    x_ref, xpad_ref, n2_ref):
    g = g_ref[...]
    sel = jnp.where(g >= 0.0, mx_ref[...], mn_ref[...])
    xb = _bn_apply2(sel, ms_ref, vs_ref, g, be_ref[...], float(E_TOTAL))
    x_ref[...] = xb
    xpad_ref[...] = jnp.concatenate(
        [xb, jnp.zeros((NB, VF - F), jnp.float32)], axis=1)
    n2_ref[...] = jnp.sum(xb * xb, axis=1, keepdims=True)


def _knn_kern(xr_ref, n2r_ref, br_ref, xf_ref, n2f_ref, bf_ref, idx_ref):
    xr = xr_ref[...]
    xf = xf_ref[...]
    dot = _dotT(xr, xf)
    d2 = n2r_ref[...] + n2f_ref[...] - 2.0 * dot
    neg = jnp.where(br_ref[...] == bf_ref[...], -d2, NEG_INF)
    iota = lax.broadcasted_iota(jnp.int32, (NB, N), 1)
    cols = []
    for _ in range(KNN):
        m = jnp.max(neg, axis=1, keepdims=True)
        cand = jnp.where(neg == m, iota, N)
        sel = jnp.min(cand, axis=1, keepdims=True)
        cols.append(sel)
        neg = jnp.where(iota == sel, NEG_INF, neg)
    idx_ref[...] = jnp.concatenate(cols, axis=1)


def _h1_slot(xi, xj3_ref, kk, d, w1_ref, b1_ref):
    xjk = xj3_ref[:, kk, :d]
    cat = jnp.concatenate([xi, xjk - xi], axis=1)
    return jnp.maximum(_dotT(cat, w1_ref[...]) + b1_ref[...], 0.0)


def _acc(ref, val):
    @pl.when(pl.program_id(0) == 0)
    def _():
        ref[...] = jnp.zeros_like(ref)

    ref[...] += val


def _edge_sum_kern(xi_ref, xj3_ref, w1_ref, b1_ref, ms_ref):
    xi = xi_ref[...]
    d = xi.shape[1]
    s = jnp.zeros((1, F), jnp.float32)
    for kk in range(KNN):
        h1k = _h1_slot(xi, xj3_ref, kk, d, w1_ref, b1_ref)
        s = s + jnp.sum(h1k, axis=0, keepdims=True)
    _acc(ms_ref, s)


def _edge_var_kern(xi_ref, xj3_ref, w1_ref, b1_ref, ms_ref, vs_ref):
    xi = xi_ref[...]
    d = xi.shape[1]
    mu = ms_ref[...] / float(E_TOTAL)
    q = jnp.zeros((1, F), jnp.float32)
    for kk in range(KNN):
        h1k = _h1_slot(xi, xj3_ref, kk, d, w1_ref, b1_ref)
        dev = h1k - mu
        q = q + jnp.sum(dev * dev, axis=0, keepdims=True)
    _acc(vs_ref, q)


def _h2_slot(xi, xj3_ref, kk, d, w1_ref, b1_ref, ms1_ref, vs1_ref, g1, be1,
             w2_ref, b2_ref):
    h1k = _h1_slot(xi, xj3_ref, kk, d, w1_ref, b1_ref)
    h1nk = _bn_apply2(h1k, ms1_ref, vs1_ref, g1, be1, float(E_TOTAL))
    return jnp.maximum(_dotT(h1nk, w2_ref[...]) + b2_ref[...], 0.0)


def _edge_out_kern(xi_ref, xj3_ref, w1_ref, b1_ref, ms1_ref, vs1_ref,
                   g1_ref, be1_ref, w2_ref, b2_ref,
                   mx_ref, mn_ref, ms2_ref):
    xi = xi_ref[...]
    d = xi.shape[1]
    g1 = g1_ref[...]
    be1 = be1_ref[...]
    mx = jnp.full((NB, F), NEG_INF, jnp.float32)
    mn = jnp.full((NB, F), POS_INF, jnp.float32)
    s2 = jnp.zeros((1, F), jnp.float32)
    for kk in range(KNN):
        zk = _h2_slot(xi, xj3_ref, kk, d, w1_ref, b1_ref, ms1_ref, vs1_ref,
                      g1, be1, w2_ref, b2_ref)
        mx = jnp.maximum(mx, zk)
        mn = jnp.minimum(mn, zk)
        s2 = s2 + jnp.sum(zk, axis=0, keepdims=True)
    mx_ref[...] = mx
    mn_ref[...] = mn
    _acc(ms2_ref, s2)


def _edge_out_var_kern(xi_ref, xj3_ref, w1_ref, b1_ref, ms1_ref, vs1_ref,
                       g1_ref, be1_ref, w2_ref, b2_ref, ms2_ref, vs2_ref):
    xi = xi_ref[...]
    d = xi.shape[1]
    g1 = g1_ref[...]
    be1 = be1_ref[...]
    mu2 = ms2_ref[...] / float(E_TOTAL)
    q2 = jnp.zeros((1, F), jnp.float32)
    for kk in range(KNN):
        zk = _h2_slot(xi, xj3_ref, kk, d, w1_ref, b1_ref, ms1_ref, vs1_ref,
                      g1, be1, w2_ref, b2_ref)
        dev = zk - mu2
        q2 = q2 + jnp.sum(dev * dev, axis=0, keepdims=True)
    _acc(vs2_ref, q2)


def _l1_kern(mx3_ref, mn3_ref, ms3_ref, vs3_ref, g3_ref, be3_ref,
             x1_ref, x2_ref, br_ref, l1w_ref, l1b_ref,
             segmx_ref, segmn_ref, stl_ref):
    g3 = g3_ref[...]
    sel3 = jnp.where(g3 >= 0.0, mx3_ref[...], mn3_ref[...])
    x3 = _bn_apply2(sel3, ms3_ref, vs3_ref, g3, be3_ref[...], float(E_TOTAL))
    cat = jnp.concatenate([x1_ref[...], x2_ref[...], x3], axis=1)
    h = jnp.maximum(_dotT(cat, l1w_ref[...]) + l1b_ref[...], 0.0)
    s = jnp.sum(h, axis=0, keepdims=True)
    q = jnp.sum(h * h, axis=0, keepdims=True)
    br = br_ref[...]
    mx_rows = []
    mn_rows = []
    for seg in range(NSEG):
        msk = br == seg
        mx_rows.append(jnp.max(jnp.where(msk, h, NEG_INF), axis=0,
                               keepdims=True))
        mn_rows.append(jnp.min(jnp.where(msk, h, POS_INF), axis=0,
                               keepdims=True))
    segmx_blk = jnp.concatenate(mx_rows, axis=0)
    segmn_blk = jnp.concatenate(mn_rows, axis=0)

    @pl.when(pl.program_id(0) == 0)
    def _():
        segmx_ref[...] = jnp.full_like(segmx_ref, NEG_INF)
        segmn_ref[...] = jnp.full_like(segmn_ref, POS_INF)
        stl_ref[...] = jnp.zeros_like(stl_ref)

    segmx_ref[...] = jnp.maximum(segmx_ref[...], segmx_blk)
    segmn_ref[...] = jnp.minimum(segmn_ref[...], segmn_blk)
    stl_ref[...] += jnp.concatenate([s, q], axis=0)


def _head_kern(segmx_ref, segmn_ref, stl_ref, g_ref, be_ref,
               m1w_ref, m1b_ref, m1g_ref, m1be_ref,
               m2w_ref, m2b_ref, m2g_ref, m2be_ref,
               m3w_ref, m3b_ref, out_ref):
    g = g_ref[...]
    sel = jnp.where(g >= 0.0, segmx_ref[...], segmn_ref[...])
    pooled = _bn_apply(sel, stl_ref, g, be_ref[...], float(N))

    def bn_exact(h, gg, be):
        mu = jnp.mean(h, axis=0, keepdims=True)
        var = jnp.mean((h - mu) ** 2, axis=0, keepdims=True)
        return gg * (h - mu) / jnp.sqrt(var + EPS) + be

    h = jnp.maximum(_dotT(pooled, m1w_ref[...]) + m1b_ref[...], 0.0)
    h = bn_exact(h, m1g_ref[...], m1be_ref[...])
    h = jnp.maximum(_dotT(h, m2w_ref[...]) + m2b_ref[...], 0.0)
    h = bn_exact(h, m2g_ref[...], m2be_ref[...])
    out_ref[...] = _dotT(h, m3w_ref[...]) + m3b_ref[...]


def _row_spec(d):
    return pl.BlockSpec((NB, d), lambda i: (i, 0))


def _full_spec(shape):
    return pl.BlockSpec(shape, lambda i: tuple(0 for _ in shape))


def _f32(shape):
    return jax.ShapeDtypeStruct(shape, jnp.float32)


def _prep0(x0):
    d = x0.shape[1]
    return pl.pallas_call(
        _prep0_kern,
        grid=(NBLK,),
        in_specs=[_row_spec(d)],
        out_specs=[_row_spec(VF), _row_spec(1)],
        out_shape=[_f32((N, VF)), _f32((N, 1))],
    )(x0)


def _next_prep(mx, mn, ms, vs, g, be):
    return pl.pallas_call(
        _next_prep_kern,
        grid=(NBLK,),
        in_specs=[_row_spec(F), _row_spec(F), _full_spec((1, F)),
                  _full_spec((1, F)), _full_spec((1, F)), _full_spec((1, F))],
        out_specs=[_row_spec(F), _row_spec(VF), _row_spec(1)],
        out_shape=[_f32((N, F)), _f32((N, VF)), _f32((N, 1))],
    )(mx, mn, ms, vs, g, be)


def _knn(x, n2, batch_col, batch_row):
    d = x.shape[1]
    return pl.pallas_call(
        _knn_kern,
        grid=(NBLK,),
        in_specs=[_row_spec(d), _row_spec(1), _row_spec(1),
                  _full_spec((N, d)), _full_spec((1, N)), _full_spec((1, N))],
        out_specs=pl.BlockSpec((NB, KNN), lambda i: (i, 0)),
        out_shape=jax.ShapeDtypeStruct((N, KNN), jnp.int32),
    )(x, n2, batch_col, x, n2.reshape(1, N), batch_row)


def _xj_spec():
    return pl.BlockSpec((NB, KNN, VF), lambda i: (i, 0, 0))


def _edge_sum(x, xj3, w1, b1):
    d = x.shape[1]
    return pl.pallas_call(
        _edge_sum_kern,
        grid=(NBLK,),
        in_specs=[_row_spec(d), _xj_spec(),
                  _full_spec((F, 2 * d)), _full_spec((1, F))],
        out_specs=_full_spec((1, F)),
        out_shape=_f32((1, F)),
    )(x, xj3, w1, b1)


def _edge_var(x, xj3, w1, b1, ms):
    d = x.shape[1]
    return pl.pallas_call(
        _edge_var_kern,
        grid=(NBLK,),
        in_specs=[_row_spec(d), _xj_spec(),
                  _full_spec((F, 2 * d)), _full_spec((1, F)),
                  _full_spec((1, F))],
        out_specs=_full_spec((1, F)),
        out_shape=_f32((1, F)),
    )(x, xj3, w1, b1, ms)


def _edge_out(x, xj3, w1, b1, ms1, vs1, g1, be1, w2, b2):
    d = x.shape[1]
    return pl.pallas_call(
        _edge_out_kern,
        grid=(NBLK,),
        in_specs=[_row_spec(d), _xj_spec(),
                  _full_spec((F, 2 * d)), _full_spec((1, F)),
                  _full_spec((1, F)), _full_spec((1, F)),
                  _full_spec((1, F)), _full_spec((1, F)),
                  _full_spec((F, F)), _full_spec((1, F))],
        out_specs=[_row_spec(F), _row_spec(F), _full_spec((1, F))],
        out_shape=[_f32((N, F)), _f32((N, F)), _f32((1, F))],
    )(x, xj3, w1, b1, ms1, vs1, g1, be1, w2, b2)


def _edge_out_var(x, xj3, w1, b1, ms1, vs1, g1, be1, w2, b2, ms2):
    d = x.shape[1]
    return pl.pallas_call(
        _edge_out_var_kern,
        grid=(NBLK,),
        in_specs=[_row_spec(d), _xj_spec(),
                  _full_spec((F, 2 * d)), _full_spec((1, F)),
                  _full_spec((1, F)), _full_spec((1, F)),
                  _full_spec((1, F)), _full_spec((1, F)),
                  _full_spec((F, F)), _full_spec((1, F)),
                  _full_spec((1, F))],
        out_specs=_full_spec((1, F)),
        out_shape=_f32((1, F)),
    )(x, xj3, w1, b1, ms1, vs1, g1, be1, w2, b2, ms2)


def _edge_conv(x_full, xpad, n2, batch_col, batch_row, w1, b1, g1, be1,
               w2, b2):
    idx = _knn(x_full, n2, batch_col, batch_row)
    xj = _sc_gather(xpad, idx.reshape(-1))
    xj3 = xj.reshape(N, KNN, VF)
    ms1 = _edge_sum(x_full, xj3, w1, b1)
    vs1 = _edge_var(x_full, xj3, w1, b1, ms1)
    mx, mn, ms2 = _edge_out(x_full, xj3, w1, b1, ms1, vs1, g1, be1, w2, b2)
    vs2 = _edge_out_var(x_full, xj3, w1, b1, ms1, vs1, g1, be1, w2, b2, ms2)
    return mx, mn, ms2, vs2


def _l1(mx3, mn3, ms3, vs3, g3, be3, x1, x2, batch_col, l1w, l1b):
    return pl.pallas_call(
        _l1_kern,
        grid=(NBLK,),
        in_specs=[_row_spec(F), _row_spec(F), _full_spec((1, F)),
                  _full_spec((1, F)),
                  _full_spec((1, F)), _full_spec((1, F)),
                  _row_spec(F), _row_spec(F), _row_spec(1),
                  _full_spec((1024, 192)), _full_spec((1, 1024))],
        out_specs=[_full_spec((NSEG, 1024)), _full_spec((NSEG, 1024)),
                   _full_spec((2, 1024))],
        out_shape=[_f32((NSEG, 1024)), _f32((NSEG, 1024)), _f32((2, 1024))],
    )(mx3, mn3, ms3, vs3, g3, be3, x1, x2, batch_col, l1w, l1b)


def _head(segmx, segmn, stl, g, be, m1w, m1b, m1g, m1be,
          m2w, m2b, m2g, m2be, m3w, m3b):
    return pl.pallas_call(
        _head_kern,
        grid=(1,),
        in_specs=[_full_spec((NSEG, 1024)), _full_spec((NSEG, 1024)),
                  _full_spec((2, 1024)), _full_spec((1, 1024)),
                  _full_spec((1, 1024)),
                  _full_spec((512, 1024)), _full_spec((1, 512)),
                  _full_spec((1, 512)), _full_spec((1, 512)),
                  _full_spec((256, 512)), _full_spec((1, 256)),
                  _full_spec((1, 256)), _full_spec((1, 256)),
                  _full_spec((40, 256)), _full_spec((1, 40))],
        out_specs=_full_spec((NSEG, 40)),
        out_shape=_f32((NSEG, 40)),
    )(segmx, segmn, stl, g, be, m1w, m1b, m1g, m1be,
      m2w, m2b, m2g, m2be, m3w, m3b)


def kernel(batch, x, pos,
           c1_W1, c1_b1, c1_g1, c1_be1, c1_W2, c1_b2, c1_g2, c1_be2,
           c2_W1, c2_b1, c2_g1, c2_be1, c2_W2, c2_b2, c2_g2, c2_be2,
           c3_W1, c3_b1, c3_g1, c3_be1, c3_W2, c3_b2, c3_g2, c3_be2,
           l1_W, l1_b, l1_g, l1_be,
           m1_W, m1_b, m1_g, m1_be,
           m2_W, m2_b, m2_g, m2_be,
           m3_W, m3_b):
    r = lambda p: p.reshape(1, -1)
    batch_col = batch.reshape(N, 1)
    batch_row = batch.reshape(1, N)

    x0 = jnp.concatenate([x, pos], axis=1)

    xpad0, n20 = _prep0(x0)
    mx1, mn1, ms1o, vs1o = _edge_conv(x0, xpad0, n20, batch_col, batch_row,
                                      c1_W1, r(c1_b1), r(c1_g1), r(c1_be1),
                                      c1_W2, r(c1_b2))

    x1, xpad1, n21 = _next_prep(mx1, mn1, ms1o, vs1o, r(c1_g2), r(c1_be2))
    mx2, mn2, ms2o, vs2o = _edge_conv(x1, xpad1, n21, batch_col, batch_row,
                                      c2_W1, r(c2_b1), r(c2_g1), r(c2_be1),
                                      c2_W2, r(c2_b2))

    x2, xpad2, n22 = _next_prep(mx2, mn2, ms2o, vs2o, r(c2_g2), r(c2_be2))
    mx3, mn3, ms3o, vs3o = _edge_conv(x2, xpad2, n22, batch_col, batch_row,
                                      c3_W1, r(c3_b1), r(c3_g1), r(c3_be1),
                                      c3_W2, r(c3_b2))

    segmx, segmn, stl = _l1(mx3, mn3, ms3o, vs3o, r(c3_g2), r(c3_be2),
                            x1, x2, batch_col, l1_W, r(l1_b))

    return _head(segmx, segmn, stl, r(l1_g), r(l1_be),
                 m1_W, r(m1_b), r(m1_g), r(m1_be),
                 m2_W, r(m2_b), r(m2_g), r(m2_be),
                 m3_W, r(m3_b))
